# baseline (speedup 1.0000x reference)
import functools

import jax
import jax.numpy as jnp
from jax import lax
from jax.experimental import pallas as pl
from jax.experimental.pallas import tpu as pltpu
from jax.experimental.pallas import tpu_sc as plsc

D = 64
NSUB = 128
KSUB = 2
CHUNK = NSUB * KSUB


def _sc_gather(ent_table, rel_table, idx_ent, idx_rel, ent_sizes, rel_sizes):
    info = plsc.get_sparse_core_info()
    nc, ns = info.num_cores, info.num_subcores
    nw = nc * ns
    out_type = [jax.ShapeDtypeStruct((n, D), jnp.bfloat16)
                for n in ent_sizes + rel_sizes]

    @functools.partial(
        pl.kernel,
        out_type=out_type,
        mesh=plsc.VectorSubcoreMesh(core_axis_name="c", subcore_axis_name="s"),
        compiler_params=pltpu.CompilerParams(use_tc_tiling_on_sc=False),
        scratch_types=[
            pltpu.VMEM((2, CHUNK), jnp.int32),
            pltpu.VMEM((2, CHUNK, D), jnp.bfloat16),
            pltpu.SemaphoreType.DMA,
            pltpu.SemaphoreType.DMA,
            pltpu.SemaphoreType.DMA,
            pltpu.SemaphoreType.DMA,
        ],
    )
    def gather_kernel(ent_hbm, rel_hbm, ie_hbm, ir_hbm, *rest):
        outs = rest[:len(out_type)]
        idx_v, rows_v, g0, g1, w0, w1 = rest[len(out_type):]
        gsem = (g0, g1)
        wsem = (w0, w1)
        wid = lax.axis_index("s") * nc + lax.axis_index("c")

        tasks = []
        base = 0
        for k, n in enumerate(ent_sizes):
            nch = n // (nw * CHUNK)
            for c in range(nch):
                off = (wid * nch + c) * CHUNK
                tasks.append((ie_hbm, base + off, outs[k], off, ent_hbm))
            base += n
        base = 0
        for k, n in enumerate(rel_sizes):
            nch = n // (nw * CHUNK)
            for c in range(nch):
                off = (wid * nch + c) * CHUNK
                tasks.append((ir_hbm, base + off,
                              outs[len(ent_sizes) + k], off, rel_hbm))
            base += n

        hg = [None, None]
        hw = [None, None]

        def fire(k):
            bi = k & 1
            idx_hbm, ioff, _, _, table = tasks[k]
            pltpu.sync_copy(idx_hbm.at[pl.ds(ioff, CHUNK)], idx_v.at[bi])
            hg[bi] = [pltpu.async_copy(
                table.at[idx_v.at[bi].at[pl.ds(j * NSUB, NSUB)]],
                rows_v.at[bi].at[pl.ds(j * NSUB, NSUB)], gsem[bi])
                for j in range(KSUB)]

        def retire(k):
            bi = k & 1
            _, _, out_hbm, ooff, _ = tasks[k]
            for h in hg[bi]:
                h.wait()
            hw[bi] = pltpu.async_copy(
                rows_v.at[bi], out_hbm.at[pl.ds(ooff, CHUNK)], wsem[bi])

        for k in range(len(tasks)):
            bi = k & 1
            if hw[bi] is not None:
                hw[bi].wait()
                hw[bi] = None
            fire(k)
            if k >= 1:
                retire(k - 1)
        retire(len(tasks) - 1)
        for bi in (0, 1):
            if hw[bi] is not None:
                hw[bi].wait()

    return gather_kernel(ent_table, rel_table, idx_ent, idx_rel)


def _softplus(x):
    return jnp.maximum(x, 0.0) + jnp.log1p(jnp.exp(-jnp.abs(x)))


def _tc_body(n_grid, total_b, nn,
             eh, er, et, ehn, ern, etn, encr, enct, p0, p1, p2,
             out_ref, acc):
    i = pl.program_id(0)
    blk = eh.shape[0]
    nd = nn * D

    @pl.when(i == 0)
    def _init():
        acc[0] = 0.0
        acc[1] = 0.0
        acc[2] = 0.0

    f32 = jnp.float32
    eh_ = eh[...].astype(f32)
    er_ = er[...].astype(f32)
    et_ = et[...].astype(f32)
    ehn_ = ehn[...].astype(f32)
    ern_ = ern[...].astype(f32)
    etn_ = etn[...].astype(f32)

    nt = encr[...].astype(f32) - enct[...].astype(f32)
    ep = p0[...].astype(f32) + p1[...].astype(f32) + p2[...].astype(f32)
    q1 = er_ - et_
    q4 = eh_ - et_
    q5 = eh_ - etn_

    def tile(x):
        return jnp.concatenate([x] * nn, axis=1)

    prods = jnp.concatenate([
        nt * nt, nt * tile(q1), nt * tile(eh_), nt * tile(ehn_),
        ep * ep, ep * tile(q4), ep * tile(er_), ep * tile(q5),
        ep * tile(ern_)], axis=0)
    seg_r = lax.broadcasted_iota(jnp.int32, (nd, nn), 0) // D
    seg_c = lax.broadcasted_iota(jnp.int32, (nd, nn), 1)
    seg = (seg_r == seg_c).astype(jnp.float32)
    red = lax.dot_general(prods, seg, (((1,), (0,)), ((), ())),
                          preferred_element_type=jnp.float32)
    r_ntnt = red[0 * blk:1 * blk]
    r_ntq1 = red[1 * blk:2 * blk]
    r_nteh = red[2 * blk:3 * blk]
    r_ntehn = red[3 * blk:4 * blk]
    r_epep = red[4 * blk:5 * blk]
    r_epq4 = red[5 * blk:6 * blk]
    r_eper = red[6 * blk:7 * blk]
    r_epq5 = red[7 * blk:8 * blk]
    r_epern = red[8 * blk:9 * blk]

    def sqnorm(x):
        return jnp.sum(x * x, axis=1, keepdims=True)

    nq1 = sqnorm(q1)
    neh = sqnorm(eh_)
    nehn = sqnorm(ehn_)
    nq4 = sqnorm(q4)
    ner = sqnorm(er_)
    nq5 = sqnorm(q5)
    nern = sqnorm(ern_)

    def pdist(v2, cross, q2):
        return jnp.sqrt(jnp.maximum(v2 + cross + q2, 0.0))

    a = pdist(r_ntnt, -2.0 * r_ntq1, nq1)
    ma = jnp.max(-a, axis=1, keepdims=True)
    ea = jnp.exp(-a - ma)
    alpha = ea / jnp.sum(ea, axis=1, keepdims=True)
    npos = pdist(r_ntnt, 2.0 * r_nteh, neh)
    nneg = pdist(r_ntnt, 2.0 * r_ntehn, nehn)
    acc[0] += jnp.sum(alpha * npos)
    acc[1] += jnp.sum(alpha * nneg)

    b = pdist(r_epep, 2.0 * r_epq4, nq4)
    mb = jnp.max(-b, axis=1, keepdims=True)
    eb = jnp.exp(-b - mb)
    beta = eb / jnp.sum(eb, axis=1, keepdims=True)
    n_ep_er = pdist(r_epep, -2.0 * r_eper, ner)
    g_pos = jnp.sum(beta * (b + n_ep_er), axis=1, keepdims=True)
    n1 = pdist(r_epep, 2.0 * r_epq5, nq5)
    n2 = pdist(r_epep, -2.0 * r_epern, nern)
    g_neg = jnp.sum(beta * (n1 + n2), axis=1, keepdims=True)
    acc[2] += jnp.sum(_softplus(-g_pos) + _softplus(g_neg))

    @pl.when(i == n_grid - 1)
    def _finalize():
        out_ref[0, 0] = acc[0]
        out_ref[0, 1] = acc[1]
        out_ref[0, 2] = acc[2]


def _tc_partials(eh, er, et, ehn, ern, etn, encr, enct, p0, p1, p2, blk=1024):
    b_total = eh.shape[0]
    nn = encr.shape[1] // D
    n_grid = b_total // blk
    vec = pl.BlockSpec((blk, D), lambda i: (i, 0))
    cub = pl.BlockSpec((blk, nn * D), lambda i: (i, 0))
    out = pl.pallas_call(
        functools.partial(_tc_body, n_grid, float(b_total), nn),
        grid=(n_grid,),
        in_specs=[vec, vec, vec, vec, vec, vec, cub, cub, cub, cub, cub],
        out_specs=pl.BlockSpec((1, 4), lambda i: (0, 0),
                               memory_space=pltpu.SMEM),
        out_shape=jax.ShapeDtypeStruct((1, 4), jnp.float32),
        scratch_shapes=[pltpu.SMEM((4,), jnp.float32)],
    )(eh, er, et, ehn, ern, etn, encr, enct, p0, p1, p2)
    return out[0, 0], out[0, 1], out[0, 2]


def _half(ent, rel2, n_rel_rows, h_batch, r_batch, t_batch, h_neg_batch,
          r_neg_batch, t_neg_batch, neighbor_context, path_context):
    b = h_batch.shape[0]
    nn = neighbor_context.shape[1]
    np_ = path_context.shape[1]
    nc_r = neighbor_context[..., 0].astype(jnp.int32)
    nc_t = neighbor_context[..., 1].astype(jnp.int32)
    pc = path_context.astype(jnp.int32)

    idx_ent = jnp.concatenate([
        h_batch.astype(jnp.int32), t_batch.astype(jnp.int32),
        h_neg_batch.astype(jnp.int32), r_neg_batch.astype(jnp.int32),
        t_neg_batch.astype(jnp.int32), nc_t.reshape(-1)])
    pc_t = jnp.transpose(pc, (2, 0, 1)).reshape(-1)
    idx_rel = jnp.concatenate([
        r_batch.astype(jnp.int32), nc_r.reshape(-1), pc_t + n_rel_rows])

    ent_sizes = [b, b, b, b, b, nn * b]
    rel_sizes = [b, nn * b, np_ * b, np_ * b, np_ * b]
    (eh, et, ehn, ern, etn, enct,
     er, encr, p0, p1, p2) = _sc_gather(ent, rel2, idx_ent, idx_rel,
                                        ent_sizes, rel_sizes)
    enct = enct.reshape(b, nn * D)
    encr = encr.reshape(b, nn * D)
    p0 = p0.reshape(b, np_ * D)
    p1 = p1.reshape(b, np_ * D)
    p2 = p2.reshape(b, np_ * D)
    return _tc_partials(eh, er, et, ehn, ern, etn, encr, enct, p0, p1, p2)


def kernel(embed_entity, embed_relation, h_batch, r_batch, t_batch,
           h_neg_batch, r_neg_batch, t_neg_batch, neighbor_context,
           path_context):
    b = h_batch.shape[0]
    ent = embed_entity.astype(jnp.bfloat16)
    rel = embed_relation.astype(jnp.bfloat16)
    n_rel_rows = rel.shape[0]
    rel2 = jnp.concatenate([rel, rel.at[0].set(0.0)], axis=0)

    h = b // 2
    gp0, gn0, pl0 = _half(ent, rel2, n_rel_rows, h_batch[:h], r_batch[:h],
                          t_batch[:h], h_neg_batch[:h], r_neg_batch[:h],
                          t_neg_batch[:h], neighbor_context[:h],
                          path_context[:h])
    gp1, gn1, pl1 = _half(ent, rel2, n_rel_rows, h_batch[h:], r_batch[h:],
                          t_batch[h:], h_neg_batch[h:], r_neg_batch[h:],
                          t_neg_batch[h:], neighbor_context[h:],
                          path_context[h:])
    return (pl0 + pl1 + float(b) * (_softplus(-(gp0 + gp1))
                                    + _softplus(gn0 + gn1)))


# speedup vs baseline: 1.8109x; 1.8109x over previous
import functools

import jax
import jax.numpy as jnp
from jax import lax
from jax.experimental import pallas as pl
from jax.experimental.pallas import tpu as pltpu
from jax.experimental.pallas import tpu_sc as plsc

D = 64
NSUB = 128
KSUB = 2
CHUNK = NSUB * KSUB


def _sc_gather(ent_table, rel_table, idx_ent, idx_rel, b, nn, np_):
    info = plsc.get_sparse_core_info()
    nc, ns = info.num_cores, info.num_subcores
    nw = nc * ns
    out_type = [jax.ShapeDtypeStruct((n, D), jnp.float32)
                for n in [b, b, b, b, b, b, nn * b, np_ * b]]

    @functools.partial(
        pl.kernel,
        out_type=out_type,
        mesh=plsc.VectorSubcoreMesh(core_axis_name="c", subcore_axis_name="s"),
        compiler_params=pltpu.CompilerParams(use_tc_tiling_on_sc=False),
        scratch_types=[
            pltpu.VMEM((2, 3, CHUNK), jnp.int32),
            pltpu.VMEM((2, 3, CHUNK, D), jnp.float32),
            pltpu.SemaphoreType.DMA,
            pltpu.SemaphoreType.DMA,
            pltpu.SemaphoreType.DMA,
            pltpu.SemaphoreType.DMA,
        ],
    )
    def gather_kernel(ent_hbm, rel_hbm, ie_hbm, ir_hbm,
                      o_eh, o_et, o_ehn, o_ern, o_etn, o_er, o_nt, o_ep,
                      idx_v, rows_v, g0, g1, w0, w1):
        gsem = (g0, g1)
        wsem = (w0, w1)
        wid = lax.axis_index("s") * nc + lax.axis_index("c")

        tasks = []
        for k, out in enumerate([o_eh, o_et, o_ehn, o_ern, o_etn]):
            off = wid * CHUNK
            tasks.append(([(ie_hbm, k * b + off, ent_hbm)], out, off, 1))
        off = wid * CHUNK
        tasks.append(([(ir_hbm, off, rel_hbm)], o_er, off, 1))
        nch = nn * b // (nw * CHUNK)
        for c in range(nch):
            off = (wid * nch + c) * CHUNK
            tasks.append(([(ir_hbm, b + off, rel_hbm),
                           (ie_hbm, 5 * b + off, ent_hbm)], o_nt, off, 2))
        pch = np_ * b // (nw * CHUNK)
        for c in range(pch):
            off = (wid * pch + c) * CHUNK
            tasks.append(([(ir_hbm, (1 + nn) * b + l * np_ * b + off,
                            rel_hbm) for l in range(3)], o_ep, off, 3))

        hg = [None, None]
        hw = [None, None]

        def fire(k):
            bi = k & 1
            srcs = tasks[k][0]
            hs = []
            for s, (idx_hbm, ioff, table) in enumerate(srcs):
                pltpu.sync_copy(idx_hbm.at[pl.ds(ioff, CHUNK)],
                                idx_v.at[bi].at[s])
                for j in range(KSUB):
                    hs.append(pltpu.async_copy(
                        table.at[idx_v.at[bi].at[s].at[pl.ds(j * NSUB,
                                                             NSUB)]],
                        rows_v.at[bi].at[s].at[pl.ds(j * NSUB, NSUB)],
                        gsem[bi]))
            hg[bi] = hs

        def retire(k):
            bi = k & 1
            _, out_hbm, ooff, n_src = tasks[k]
            for h in hg[bi]:
                h.wait()
            if n_src > 1:
                b0 = rows_v.at[bi].at[0]
                b1 = rows_v.at[bi].at[1]
                b2 = rows_v.at[bi].at[2]

                def combine(r, carry):
                    for c4 in range(D // 16):
                        sl = pl.ds(c4 * 16, 16)
                        if n_src == 2:
                            b0[r, sl] = b0[r, sl] - b1[r, sl]
                        else:
                            b0[r, sl] = b0[r, sl] + b1[r, sl] + b2[r, sl]
                    return carry
                lax.fori_loop(0, CHUNK, combine, 0)
            hw[bi] = pltpu.async_copy(
                rows_v.at[bi].at[0], out_hbm.at[pl.ds(ooff, CHUNK)],
                wsem[bi])

        for k in range(len(tasks)):
            bi = k & 1
            if hw[bi] is not None:
                hw[bi].wait()
                hw[bi] = None
            fire(k)
            if k >= 1:
                retire(k - 1)
        retire(len(tasks) - 1)
        for bi in (0, 1):
            if hw[bi] is not None:
                hw[bi].wait()

    return gather_kernel(ent_table, rel_table, idx_ent, idx_rel)


def _softplus(x):
    return jnp.maximum(x, 0.0) + jnp.log1p(jnp.exp(-jnp.abs(x)))


def _tc_body(n_grid, total_b, nn,
             eh, er, et, ehn, ern, etn, nt_in, ep_in,
             out_ref, acc):
    i = pl.program_id(0)
    blk = eh.shape[0]
    nd = nn * D

    @pl.when(i == 0)
    def _init():
        acc[0] = 0.0
        acc[1] = 0.0
        acc[2] = 0.0

    eh_ = eh[...]
    er_ = er[...]
    et_ = et[...]
    ehn_ = ehn[...]
    ern_ = ern[...]
    etn_ = etn[...]

    nt = nt_in[...]
    ep = ep_in[...]
    q1 = er_ - et_
    q4 = eh_ - et_
    q5 = eh_ - etn_

    def tile(x):
        return jnp.concatenate([x] * nn, axis=1)

    prods = jnp.concatenate([
        nt * nt, nt * tile(q1), nt * tile(eh_), nt * tile(ehn_),
        ep * ep, ep * tile(q4), ep * tile(er_), ep * tile(q5),
        ep * tile(ern_)], axis=0)
    seg_r = lax.broadcasted_iota(jnp.int32, (nd, nn), 0) // D
    seg_c = lax.broadcasted_iota(jnp.int32, (nd, nn), 1)
    seg = (seg_r == seg_c).astype(jnp.float32)
    red = lax.dot_general(prods, seg, (((1,), (0,)), ((), ())),
                          preferred_element_type=jnp.float32)
    r_ntnt = red[0 * blk:1 * blk]
    r_ntq1 = red[1 * blk:2 * blk]
    r_nteh = red[2 * blk:3 * blk]
    r_ntehn = red[3 * blk:4 * blk]
    r_epep = red[4 * blk:5 * blk]
    r_epq4 = red[5 * blk:6 * blk]
    r_eper = red[6 * blk:7 * blk]
    r_epq5 = red[7 * blk:8 * blk]
    r_epern = red[8 * blk:9 * blk]

    def sqnorm(x):
        return jnp.sum(x * x, axis=1, keepdims=True)

    nq1 = sqnorm(q1)
    neh = sqnorm(eh_)
    nehn = sqnorm(ehn_)
    nq4 = sqnorm(q4)
    ner = sqnorm(er_)
    nq5 = sqnorm(q5)
    nern = sqnorm(ern_)

    def pdist(v2, cross, q2):
        return jnp.sqrt(jnp.maximum(v2 + cross + q2, 0.0))

    a = pdist(r_ntnt, -2.0 * r_ntq1, nq1)
    ma = jnp.max(-a, axis=1, keepdims=True)
    ea = jnp.exp(-a - ma)
    alpha = ea / jnp.sum(ea, axis=1, keepdims=True)
    npos = pdist(r_ntnt, 2.0 * r_nteh, neh)
    nneg = pdist(r_ntnt, 2.0 * r_ntehn, nehn)
    acc[0] += jnp.sum(alpha * npos)
    acc[1] += jnp.sum(alpha * nneg)

    b = pdist(r_epep, 2.0 * r_epq4, nq4)
    mb = jnp.max(-b, axis=1, keepdims=True)
    eb = jnp.exp(-b - mb)
    beta = eb / jnp.sum(eb, axis=1, keepdims=True)
    n_ep_er = pdist(r_epep, -2.0 * r_eper, ner)
    g_pos = jnp.sum(beta * (b + n_ep_er), axis=1, keepdims=True)
    n1 = pdist(r_epep, 2.0 * r_epq5, nq5)
    n2 = pdist(r_epep, -2.0 * r_epern, nern)
    g_neg = jnp.sum(beta * (n1 + n2), axis=1, keepdims=True)
    acc[2] += jnp.sum(_softplus(-g_pos) + _softplus(g_neg))

    @pl.when(i == n_grid - 1)
    def _finalize():
        out_ref[0, 0] = acc[0]
        out_ref[0, 1] = acc[1]
        out_ref[0, 2] = acc[2]


def _tc_partials(eh, er, et, ehn, ern, etn, nt, ep, blk=1024):
    b_total = eh.shape[0]
    nn = nt.shape[1] // D
    n_grid = b_total // blk
    vec = pl.BlockSpec((blk, D), lambda i: (i, 0))
    cub = pl.BlockSpec((blk, nn * D), lambda i: (i, 0))
    out = pl.pallas_call(
        functools.partial(_tc_body, n_grid, float(b_total), nn),
        grid=(n_grid,),
        in_specs=[vec, vec, vec, vec, vec, vec, cub, cub],
        out_specs=pl.BlockSpec((1, 4), lambda i: (0, 0),
                               memory_space=pltpu.SMEM),
        out_shape=jax.ShapeDtypeStruct((1, 4), jnp.float32),
        scratch_shapes=[pltpu.SMEM((4,), jnp.float32)],
    )(eh, er, et, ehn, ern, etn, nt, ep)
    return out[0, 0], out[0, 1], out[0, 2]


def _half(ent, rel2, n_rel_rows, h_batch, r_batch, t_batch, h_neg_batch,
          r_neg_batch, t_neg_batch, neighbor_context, path_context):
    b = h_batch.shape[0]
    nn = neighbor_context.shape[1]
    np_ = path_context.shape[1]
    nc_r = neighbor_context[..., 0].astype(jnp.int32)
    nc_t = neighbor_context[..., 1].astype(jnp.int32)
    pc = path_context.astype(jnp.int32)

    idx_ent = jnp.concatenate([
        h_batch.astype(jnp.int32), t_batch.astype(jnp.int32),
        h_neg_batch.astype(jnp.int32), r_neg_batch.astype(jnp.int32),
        t_neg_batch.astype(jnp.int32), nc_t.reshape(-1)])
    pc_t = jnp.transpose(pc, (2, 0, 1)).reshape(-1)
    idx_rel = jnp.concatenate([
        r_batch.astype(jnp.int32), nc_r.reshape(-1), pc_t + n_rel_rows])

    (eh, et, ehn, ern, etn, er, nt, ep) = _sc_gather(
        ent, rel2, idx_ent, idx_rel, b, nn, np_)
    nt = nt.reshape(b, nn * D)
    ep = ep.reshape(b, np_ * D)
    return _tc_partials(eh, er, et, ehn, ern, etn, nt, ep)


def kernel(embed_entity, embed_relation, h_batch, r_batch, t_batch,
           h_neg_batch, r_neg_batch, t_neg_batch, neighbor_context,
           path_context):
    b = h_batch.shape[0]
    ent = embed_entity.astype(jnp.float32)
    rel = embed_relation.astype(jnp.float32)
    n_rel_rows = rel.shape[0]
    rel2 = jnp.concatenate([rel, rel.at[0].set(0.0)], axis=0)

    h = b // 2
    gp0, gn0, pl0 = _half(ent, rel2, n_rel_rows, h_batch[:h], r_batch[:h],
                          t_batch[:h], h_neg_batch[:h], r_neg_batch[:h],
                          t_neg_batch[:h], neighbor_context[:h],
                          path_context[:h])
    gp1, gn1, pl1 = _half(ent, rel2, n_rel_rows, h_batch[h:], r_batch[h:],
                          t_batch[h:], h_neg_batch[h:], r_neg_batch[h:],
                          t_neg_batch[h:], neighbor_context[h:],
                          path_context[h:])
    return (pl0 + pl1 + float(b) * (_softplus(-(gp0 + gp1))
                                    + _softplus(gn0 + gn1)))
